# Initial kernel scaffold; baseline (speedup 1.0000x reference)
import jax
import jax.numpy as jnp
from jax.experimental import pallas as pl


def kernel(x, W_enc, b_enc, W_dec, b_dec):
    raise NotImplementedError("write your pallas kernel here")


# speedup vs baseline: 1.0087x; 1.0087x over previous
import functools

import jax
import jax.numpy as jnp
from jax.experimental import pallas as pl
from jax.experimental.pallas import tpu as pltpu

N = 2048
D = 2048
T = 4
TILE = D // T
L = 4096
LTOT = T * L
K = 128

ENC_BR = 128
DEC_BR = 128


def _encode_body(x_ref, w_ref, be_ref, bd_ref, out_ref):
    xb = x_ref[...]
    be = be_ref[...]
    bd = bd_ref[...]
    for t in range(T):
        xt = xb[:, t * TILE:(t + 1) * TILE] - bd[t][None, :]
        pre = jnp.dot(xt, w_ref[t], preferred_element_type=jnp.float32)
        pre = pre + be[t][None, :]
        out_ref[:, t * L:(t + 1) * L] = jnp.maximum(pre, 0.0)


def _encode(x, W_enc_t, b_enc, b_dec):
    grid = (N // ENC_BR,)
    return pl.pallas_call(
        _encode_body,
        grid=grid,
        in_specs=[
            pl.BlockSpec((ENC_BR, D), lambda i: (i, 0)),
            pl.BlockSpec((T, TILE, L), lambda i: (0, 0, 0)),
            pl.BlockSpec((T, L), lambda i: (0, 0)),
            pl.BlockSpec((T, TILE), lambda i: (0, 0)),
        ],
        out_specs=pl.BlockSpec((ENC_BR, LTOT), lambda i: (i, 0)),
        out_shape=jax.ShapeDtypeStruct((N, LTOT), jnp.float32),
    )(x, W_enc_t, b_enc, b_dec)


def _decode_body(dense_ref, w_ref, bd_ref, x_ref, out_ref, e2_ref, tv_ref,
                 cs_ref):
    i = pl.program_id(0)
    nsteps = pl.num_programs(0)
    db = dense_ref[...]
    xb = x_ref[...]
    bd = bd_ref[...]
    outs = []
    for t in range(T):
        o = jnp.dot(db[:, t * L:(t + 1) * L], w_ref[t],
                    preferred_element_type=jnp.float32)
        outs.append(o + bd[t][None, :])
    out = jnp.concatenate(outs, axis=1)
    out_ref[...] = out

    e = xb - out

    @pl.when(i == 0)
    def _():
        e2_ref[...] = jnp.zeros_like(e2_ref)
        tv_ref[...] = jnp.zeros_like(tv_ref)
        cs_ref[...] = jnp.zeros_like(cs_ref)

    e2_ref[...] += jnp.sum(e * e)[None, None]
    tv_ref[...] += jnp.sum(xb * xb)[None, None]
    cs_ref[...] += jnp.sum(xb, axis=0, keepdims=True)

    @pl.when(i == nsteps - 1)
    def _():
        cs = cs_ref[...]
        tv_ref[...] = tv_ref[...] - jnp.sum(cs * cs)[None, None] / N


def _decode(dense, W_dec, b_dec, x):
    grid = (N // DEC_BR,)
    out_shapes = (
        jax.ShapeDtypeStruct((N, D), jnp.float32),
        jax.ShapeDtypeStruct((1, 1), jnp.float32),
        jax.ShapeDtypeStruct((1, 1), jnp.float32),
        jax.ShapeDtypeStruct((1, D), jnp.float32),
    )
    return pl.pallas_call(
        _decode_body,
        grid=grid,
        in_specs=[
            pl.BlockSpec((DEC_BR, LTOT), lambda i: (i, 0)),
            pl.BlockSpec((T, L, TILE), lambda i: (0, 0, 0)),
            pl.BlockSpec((T, TILE), lambda i: (0, 0)),
            pl.BlockSpec((DEC_BR, D), lambda i: (i, 0)),
        ],
        out_specs=(
            pl.BlockSpec((DEC_BR, D), lambda i: (i, 0)),
            pl.BlockSpec((1, 1), lambda i: (0, 0)),
            pl.BlockSpec((1, 1), lambda i: (0, 0)),
            pl.BlockSpec((1, D), lambda i: (0, 0)),
        ),
        out_shape=out_shapes,
    )(dense, W_dec, b_dec, x)


def kernel(x, W_enc, b_enc, W_dec, b_dec):
    W_enc_t = W_enc.transpose(0, 2, 1)
    pre = _encode(x, W_enc_t, b_enc, b_dec)

    top_acts, top_indices = jax.lax.top_k(pre, K)

    rows = jnp.arange(N)[:, None]
    dense = jnp.zeros((N, LTOT), jnp.float32).at[rows, top_indices].set(top_acts)

    sae_out, e2, tv, _cs = _decode(dense, W_dec, b_dec, x)
    fvu = e2[0, 0] / tv[0, 0]
    auxk_loss = jnp.asarray(0.0, dtype=jnp.float32)
    multi_topk_fvu = jnp.asarray(0.0, dtype=jnp.float32)
    return sae_out, top_acts, top_indices, fvu, auxk_loss, multi_topk_fvu
